# baseline (speedup 1.0000x reference)
import functools

import jax
import jax.numpy as jnp
from jax import lax
from jax.experimental import pallas as pl
from jax.experimental.pallas import tpu as pltpu
from jax.experimental.pallas import tpu_sc as plsc

_B = 16384
_F = 128
_L = 16
_C = 128


def _mf_body(u_idx, i_idx, u_emb, i_emb, u_bias, i_bias, out,
             uidx_v, iidx_v, urows_v, irows_v, ub_v, ib_v, dots_v, out_v, sem,
             *, rows_per_worker, num_cores):
    wid = lax.axis_index("s") * num_cores + lax.axis_index("c")
    lane_iota = lax.iota(jnp.int32, _L)

    for c in range(rows_per_worker // _C):
        base = wid * rows_per_worker + c * _C
        pltpu.sync_copy(u_idx.at[pl.ds(base, _C)], uidx_v)
        pltpu.sync_copy(i_idx.at[pl.ds(base, _C)], iidx_v)
        cps = [
            pltpu.async_copy(u_emb.at[uidx_v], urows_v, sem),
            pltpu.async_copy(i_emb.at[iidx_v], irows_v, sem),
            pltpu.async_copy(u_bias.at[uidx_v], ub_v, sem),
            pltpu.async_copy(i_bias.at[iidx_v], ib_v, sem),
        ]
        for cp in cps:
            cp.wait()

        def zero_body(g, _):
            dots_v[pl.ds(g * _L, _L)] = jnp.zeros((_L,), jnp.float32)
            return 0

        lax.fori_loop(0, _C // _L, zero_body, 0)

        def row_group_body(g, _):
            gbase = g * _L
            for r in range(_L):
                row = gbase + r
                acc = urows_v[row, pl.ds(0, _L)] * irows_v[row, pl.ds(0, _L)]
                for k in range(1, _F // _L):
                    acc += (urows_v[row, pl.ds(k * _L, _L)]
                            * irows_v[row, pl.ds(k * _L, _L)])
                plsc.addupdate_scatter(dots_v, [jnp.full((_L,), row, jnp.int32)],
                                       acc)
            gslice = pl.ds(gbase, _L)
            pred = dots_v[gslice] + ub_v[gslice] + ib_v[gslice]
            out_v[gslice] = 1.0 / (1.0 + jnp.exp(-pred))
            return 0

        lax.fori_loop(0, _C // _L, row_group_body, 0)
        pltpu.sync_copy(out_v, out.at[pl.ds(base, _C)])


@functools.cache
def _build():
    info = plsc.get_sparse_core_info()
    num_workers = info.num_cores * info.num_subcores
    rows_per_worker = _B // num_workers
    mesh = plsc.VectorSubcoreMesh(core_axis_name="c", subcore_axis_name="s")
    body = functools.partial(_mf_body, rows_per_worker=rows_per_worker,
                             num_cores=info.num_cores)
    return pl.kernel(
        body,
        out_type=jax.ShapeDtypeStruct((_B,), jnp.float32),
        mesh=mesh,
        compiler_params=pltpu.CompilerParams(needs_layout_passes=False),
        scratch_types=[
            pltpu.VMEM((_C,), jnp.int32),
            pltpu.VMEM((_C,), jnp.int32),
            pltpu.VMEM((_C, _F), jnp.float32),
            pltpu.VMEM((_C, _F), jnp.float32),
            pltpu.VMEM((_C,), jnp.float32),
            pltpu.VMEM((_C,), jnp.float32),
            pltpu.VMEM((_C,), jnp.float32),
            pltpu.VMEM((_C,), jnp.float32),
            pltpu.SemaphoreType.DMA,
        ],
    )


def kernel(u_idx, i_idx, u_emb, i_emb, u_bias, i_bias):
    return _build()(u_idx.astype(jnp.int32), i_idx.astype(jnp.int32),
                    u_emb, i_emb,
                    u_bias.reshape(-1), i_bias.reshape(-1))
